# baseline (speedup 1.0000x reference)
import dataclasses

import jax
import jax.numpy as jnp
from jax import lax
from jax.experimental import pallas as pl
from jax.experimental.pallas import tpu as pltpu
from jax.experimental.pallas import tpu_sc as plsc

_VOCAB = 3
_EMBED = 2
_UNITS = 4
_LANES = 16

_B = 16384
_H = 200
_N = _B * _H
_NJG = 8
_NIQ = 4
_JB = _H // _NJG
_IB = _B // _NIQ
_IC = 512
_NCHUNKS = _IB // _IC
_JSTRIDE = _B * _UNITS


def _body(idxt_hbm, params_hbm, out_hbm, params_v, idx0, idx1, out0, out1,
          si0, si1, so0, so1):
    c = lax.axis_index("c")
    s = lax.axis_index("s")
    wid = s * 2 + c
    jg = wid % _NJG
    iq = wid // _NJG
    j0 = jg * _JB
    i0q = iq * _IB

    pltpu.sync_copy(params_hbm, params_v)

    def splat(v):
        return jnp.full((_LANES,), v, jnp.int32)

    fused = []
    for v in range(_VOCAB):
        t0 = plsc.load_gather(params_v, [splat(1 + v * _EMBED + 0)])
        t1 = plsc.load_gather(params_v, [splat(1 + v * _EMBED + 1)])
        row = []
        for d in range(_UNITS):
            w0 = plsc.load_gather(params_v, [splat(7 + 0 * _UNITS + d)])
            w1 = plsc.load_gather(params_v, [splat(7 + 1 * _UNITS + d)])
            bd = plsc.load_gather(params_v, [splat(15 + d)])
            row.append(t0 * w0 + t1 * w1 + bd)
        fused.append(row)

    idx_bufs = (idx0, idx1)
    out_bufs = (out0, out1)
    isems = (si0, si1)
    osems = (so0, so1)

    def start_in(ci, b):
        i0 = i0q + ci * _IC
        ib = idx_bufs[b]
        for jj in range(_JB):
            src = idxt_hbm.at[pl.ds((j0 + jj) * _B + i0, _IC)]
            pltpu.async_copy(src, ib.at[pl.ds(jj * _IC, _IC)], isems[b])

    def start_out(ci, b):
        ob = out_bufs[b]
        o0 = iq * (_IB * _UNITS) + ci * (_IC * _UNITS)
        for jj in range(_JB):
            dst = out_hbm.at[pl.ds((j0 + jj) * _JSTRIDE + o0, _IC * _UNITS)]
            pltpu.async_copy(
                ob.at[pl.ds(jj * _IC * _UNITS, _IC * _UNITS)], dst, osems[b])

    def wait_in(b):
        pltpu.make_async_copy(
            idxt_hbm.at[pl.ds(0, _JB * _IC)], idx_bufs[b], isems[b]).wait()

    def wait_out(b):
        pltpu.make_async_copy(
            out_bufs[b], out_hbm.at[pl.ds(0, _JB * _IC * _UNITS)], osems[b]).wait()

    def compute(ci, b):
        ib = idx_bufs[b]
        ob = out_bufs[b]

        def jj_body(jj, carry):
            ibase = jj * _IC
            obase = jj * (_IC * _UNITS)
            for k in range(_IC // _LANES):
                idxv = ib[pl.ds(ibase + k * _LANES, _LANES)]
                m0 = idxv == 0
                m1 = idxv == 1
                off = (k // 8) * 512 + (k % 8) * _LANES
                for d in range(_UNITS):
                    val = jnp.where(m0, fused[0][d],
                                    jnp.where(m1, fused[1][d], fused[2][d]))
                    ob[pl.ds(obase + off + d * 128, _LANES)] = val
            return carry

        lax.fori_loop(0, _JB, jj_body, 0)

    start_in(0, 0)
    start_in(1, 1)
    for ci in range(2):
        wait_in(ci)
        compute(ci, ci)
        start_out(ci, ci)
        start_in(ci + 2, ci)

    def pair(g, carry):
        for b in range(2):
            ci = 2 * g + b
            wait_in(b)
            wait_out(b)
            compute(ci, b)
            start_out(ci, b)
            start_in(ci + 2, b)
        return carry

    lax.fori_loop(1, _NCHUNKS // 2 - 1, pair, 0)

    for ci in range(_NCHUNKS - 2, _NCHUNKS):
        b = ci % 2
        wait_in(b)
        wait_out(b)
        compute(ci, b)
        start_out(ci, b)
    wait_out(0)
    wait_out(1)


@jax.jit
def _sc_call(idxt, params):
    mesh = plsc.VectorSubcoreMesh(core_axis_name="c", subcore_axis_name="s")
    cp = pltpu.CompilerParams()
    if "needs_layout_passes" in pltpu.CompilerParams.__dataclass_fields__:
        cp = dataclasses.replace(cp, needs_layout_passes=False)
    return pl.kernel(
        _body,
        out_type=jax.ShapeDtypeStruct((_N * _UNITS,), jnp.float32),
        name="fused_embed_dense_sc",
        mesh=mesh,
        compiler_params=cp,
        scratch_types=[
            pltpu.VMEM((32,), jnp.float32),
            pltpu.VMEM((_JB * _IC,), jnp.int32),
            pltpu.VMEM((_JB * _IC,), jnp.int32),
            pltpu.VMEM((_JB * _IC * _UNITS,), jnp.float32),
            pltpu.VMEM((_JB * _IC * _UNITS,), jnp.float32),
            pltpu.SemaphoreType.DMA,
            pltpu.SemaphoreType.DMA,
            pltpu.SemaphoreType.DMA,
            pltpu.SemaphoreType.DMA,
        ],
    )(idxt, params)


def kernel(inputs, table, W, b):
    idxt = inputs.T.astype(jnp.int32).reshape(-1)
    params = jnp.concatenate(
        [
            jnp.zeros((1,), jnp.float32),
            table.reshape(-1).astype(jnp.float32),
            W.reshape(-1).astype(jnp.float32),
            b.reshape(-1).astype(jnp.float32),
            jnp.zeros((31 - _VOCAB * _EMBED - _EMBED * _UNITS - _UNITS,), jnp.float32),
        ]
    )
    out = _sc_call(idxt, params)
    out = out.reshape(_H, _B // 128, _UNITS, 128).transpose(1, 3, 0, 2)
    return out.reshape(inputs.shape + (_UNITS,))


# speedup vs baseline: 1.1149x; 1.1149x over previous
import dataclasses

import jax
import jax.numpy as jnp
from jax import lax
from jax.experimental import pallas as pl
from jax.experimental.pallas import tpu as pltpu
from jax.experimental.pallas import tpu_sc as plsc

_VOCAB = 3
_EMBED = 2
_UNITS = 4
_LANES = 16

_B = 16384
_H = 200
_N = _B * _H
_NJG = 8
_NIQ = 4
_JB = _H // _NJG
_IB = _B // _NIQ
_IC = 512
_NCHUNKS = _IB // _IC
_JSTRIDE = _B * _UNITS


def _body(idxt_hbm, params_hbm, out_hbm, params_v, idx0, idx1, out0, out1,
          si0, si1, so0, so1):
    c = lax.axis_index("c")
    s = lax.axis_index("s")
    wid = s * 2 + c
    jg = wid % _NJG
    iq = wid // _NJG
    j0 = jg * _JB
    i0q = iq * _IB

    pltpu.sync_copy(params_hbm, params_v)

    def splat(v):
        return jnp.full((_LANES,), v, jnp.int32)

    fused = []
    for v in range(_VOCAB):
        t0 = plsc.load_gather(params_v, [splat(1 + v * _EMBED + 0)])
        t1 = plsc.load_gather(params_v, [splat(1 + v * _EMBED + 1)])
        row = []
        for d in range(_UNITS):
            w0 = plsc.load_gather(params_v, [splat(7 + 0 * _UNITS + d)])
            w1 = plsc.load_gather(params_v, [splat(7 + 1 * _UNITS + d)])
            bd = plsc.load_gather(params_v, [splat(15 + d)])
            row.append(t0 * w0 + t1 * w1 + bd)
        fused.append(row)

    idx_bufs = (idx0, idx1)
    out_bufs = (out0, out1)
    isems = (si0, si1)
    osems = (so0, so1)

    def start_in(ci, b):
        it0 = iq * (_IB // 128) + ci * (_IC // 128)
        ib = idx_bufs[b]
        for jj in range(_JB):
            jfull = j0 + jj
            jbase = (jfull // 8) * (_B * 8) + (jfull % 8) * 128
            for t in range(_IC // 128):
                src = idxt_hbm.at[pl.ds(jbase + (it0 + t) * 1024, 128)]
                pltpu.async_copy(
                    src, ib.at[pl.ds(jj * _IC + t * 128, 128)], isems[b])

    def start_out(ci, b):
        ob = out_bufs[b]
        o0 = iq * (_IB * _UNITS) + ci * (_IC * _UNITS)
        for jj in range(_JB):
            dst = out_hbm.at[pl.ds((j0 + jj) * _JSTRIDE + o0, _IC * _UNITS)]
            pltpu.async_copy(
                ob.at[pl.ds(jj * _IC * _UNITS, _IC * _UNITS)], dst, osems[b])

    def wait_in(b):
        pltpu.make_async_copy(
            idxt_hbm.at[pl.ds(0, _JB * _IC)], idx_bufs[b], isems[b]).wait()

    def wait_out(b):
        pltpu.make_async_copy(
            out_bufs[b], out_hbm.at[pl.ds(0, _JB * _IC * _UNITS)], osems[b]).wait()

    def compute(ci, b):
        ib = idx_bufs[b]
        ob = out_bufs[b]

        def jj_body(jj, carry):
            ibase = jj * _IC
            obase = jj * (_IC * _UNITS)
            for k in range(_IC // _LANES):
                idxv = ib[pl.ds(ibase + k * _LANES, _LANES)]
                m0 = idxv == 0
                m1 = idxv == 1
                off = (k // 8) * 512 + (k % 8) * _LANES
                for d in range(_UNITS):
                    val = jnp.where(m0, fused[0][d],
                                    jnp.where(m1, fused[1][d], fused[2][d]))
                    ob[pl.ds(obase + off + d * 128, _LANES)] = val
            return carry

        lax.fori_loop(0, _JB, jj_body, 0)

    start_in(0, 0)
    start_in(1, 1)
    for ci in range(2):
        wait_in(ci)
        compute(ci, ci)
        start_out(ci, ci)
        start_in(ci + 2, ci)

    def pair(g, carry):
        for b in range(2):
            ci = 2 * g + b
            wait_in(b)
            wait_out(b)
            compute(ci, b)
            start_out(ci, b)
            start_in(ci + 2, b)
        return carry

    lax.fori_loop(1, _NCHUNKS // 2 - 1, pair, 0)

    for ci in range(_NCHUNKS - 2, _NCHUNKS):
        b = ci % 2
        wait_in(b)
        wait_out(b)
        compute(ci, b)
        start_out(ci, b)
    wait_out(0)
    wait_out(1)


@jax.jit
def _sc_call(idxt, params):
    mesh = plsc.VectorSubcoreMesh(core_axis_name="c", subcore_axis_name="s")
    cp = pltpu.CompilerParams()
    if "needs_layout_passes" in pltpu.CompilerParams.__dataclass_fields__:
        cp = dataclasses.replace(cp, needs_layout_passes=False)
    return pl.kernel(
        _body,
        out_type=jax.ShapeDtypeStruct((_N * _UNITS,), jnp.float32),
        name="fused_embed_dense_sc",
        mesh=mesh,
        compiler_params=cp,
        scratch_types=[
            pltpu.VMEM((32,), jnp.float32),
            pltpu.VMEM((_JB * _IC,), jnp.int32),
            pltpu.VMEM((_JB * _IC,), jnp.int32),
            pltpu.VMEM((_JB * _IC * _UNITS,), jnp.float32),
            pltpu.VMEM((_JB * _IC * _UNITS,), jnp.float32),
            pltpu.SemaphoreType.DMA,
            pltpu.SemaphoreType.DMA,
            pltpu.SemaphoreType.DMA,
            pltpu.SemaphoreType.DMA,
        ],
    )(idxt, params)


def kernel(inputs, table, W, b):
    idxt = (inputs.astype(jnp.int32)
            .reshape(_B // 128, 128, _H // 8, 8)
            .transpose(2, 0, 3, 1)
            .reshape(-1))
    params = jnp.concatenate(
        [
            jnp.zeros((1,), jnp.float32),
            table.reshape(-1).astype(jnp.float32),
            W.reshape(-1).astype(jnp.float32),
            b.reshape(-1).astype(jnp.float32),
            jnp.zeros((31 - _VOCAB * _EMBED - _EMBED * _UNITS - _UNITS,), jnp.float32),
        ]
    )
    out = _sc_call(idxt, params)
    out = out.reshape(_H, _B // 128, _UNITS, 128).transpose(1, 3, 0, 2)
    return out.reshape(inputs.shape + (_UNITS,))


# speedup vs baseline: 1.2254x; 1.0990x over previous
import dataclasses

import jax
import jax.numpy as jnp
from jax import lax
from jax.experimental import pallas as pl
from jax.experimental.pallas import tpu as pltpu
from jax.experimental.pallas import tpu_sc as plsc

_VOCAB = 3
_EMBED = 2
_UNITS = 4
_LANES = 16

_B = 16384
_H = 200
_N = _B * _H
_NJG = 8
_NIQ = 4
_JB = _H // _NJG
_IB = _B // _NIQ
_IC = 256
_NCHUNKS = _IB // _IC
_JT = 4
_JSTRIDE = _B * _UNITS


def _body(idxt_hbm, params_hbm, out_hbm, params_v, idx0, idx1, out0, out1,
          si0, si1, so0, so1):
    c = lax.axis_index("c")
    s = lax.axis_index("s")
    wid = s * 2 + c
    jg = wid % _NJG
    iq = wid // _NJG
    j0 = jg * _JB
    i0q = iq * _IB

    pltpu.sync_copy(params_hbm, params_v)

    def splat(v):
        return jnp.full((_LANES,), v, jnp.int32)

    fused = []
    for v in range(_VOCAB):
        t0 = plsc.load_gather(params_v, [splat(1 + v * _EMBED + 0)])
        t1 = plsc.load_gather(params_v, [splat(1 + v * _EMBED + 1)])
        row = []
        for d in range(_UNITS):
            w0 = plsc.load_gather(params_v, [splat(7 + 0 * _UNITS + d)])
            w1 = plsc.load_gather(params_v, [splat(7 + 1 * _UNITS + d)])
            bd = plsc.load_gather(params_v, [splat(15 + d)])
            row.append(t0 * w0 + t1 * w1 + bd)
        fused.append(row)

    idx_bufs = (idx0, idx1)
    out_bufs = (out0, out1)
    isems = (si0, si1)
    osems = (so0, so1)

    jhi0 = j0 // 8

    def start_in(ci, b):
        it0 = iq * (_IB // 128) + ci * (_IC // 128)
        ib = idx_bufs[b]
        for a in range(_JT):
            for t in range(_IC // 128):
                src = idxt_hbm.at[pl.ds((jhi0 + a) * (_B * 8) + (it0 + t) * 1024, 1024)]
                pltpu.async_copy(
                    src,
                    ib.at[pl.ds((a * (_IC // 128) + t) * 1024, 1024)],
                    isems[b],
                )

    def start_out(ci, b):
        ob = out_bufs[b]
        o0 = iq * (_IB * _UNITS) + ci * (_IC * _UNITS)
        for jj in range(_JB):
            dst = out_hbm.at[pl.ds((j0 + jj) * _JSTRIDE + o0, _IC * _UNITS)]
            pltpu.async_copy(
                ob.at[pl.ds(jj * _IC * _UNITS, _IC * _UNITS)], dst, osems[b])

    def wait_in(b):
        pltpu.make_async_copy(
            idxt_hbm.at[pl.ds(0, _JT * 8 * _IC)], idx_bufs[b], isems[b]).wait()

    def wait_out(b):
        pltpu.make_async_copy(
            out_bufs[b], out_hbm.at[pl.ds(0, _JB * _IC * _UNITS)], osems[b]).wait()

    def compute(ci, b):
        ib = idx_bufs[b]
        ob = out_bufs[b]

        def jj_body(jj, carry):
            jfull = j0 + jj
            a = jfull // 8 - jhi0
            jlo = jfull % 8
            obase = jj * (_IC * _UNITS)
            for k in range(_IC // _LANES):
                t = k // 8
                ibase = ((a * (_IC // 128) + t) * 8 + jlo) * 128
                idxv = ib[pl.ds(ibase + (k % 8) * _LANES, _LANES)]
                m0 = idxv == 0
                m1 = idxv == 1
                off = t * 512 + (k % 8) * _LANES
                for d in range(_UNITS):
                    val = jnp.where(m0, fused[0][d],
                                    jnp.where(m1, fused[1][d], fused[2][d]))
                    ob[pl.ds(obase + off + d * 128, _LANES)] = val
            return carry

        lax.fori_loop(0, _JB, jj_body, 0)

    start_in(0, 0)
    start_in(1, 1)
    for ci in range(2):
        wait_in(ci)
        compute(ci, ci)
        start_out(ci, ci)
        start_in(ci + 2, ci)

    def pair(g, carry):
        for b in range(2):
            ci = 2 * g + b
            wait_in(b)
            wait_out(b)
            compute(ci, b)
            start_out(ci, b)
            start_in(ci + 2, b)
        return carry

    lax.fori_loop(1, _NCHUNKS // 2 - 1, pair, 0)

    for ci in range(_NCHUNKS - 2, _NCHUNKS):
        b = ci % 2
        wait_in(b)
        wait_out(b)
        compute(ci, b)
        start_out(ci, b)
    wait_out(0)
    wait_out(1)


@jax.jit
def _sc_call(idxt, params):
    mesh = plsc.VectorSubcoreMesh(core_axis_name="c", subcore_axis_name="s")
    cp = pltpu.CompilerParams()
    if "needs_layout_passes" in pltpu.CompilerParams.__dataclass_fields__:
        cp = dataclasses.replace(cp, needs_layout_passes=False)
    return pl.kernel(
        _body,
        out_type=jax.ShapeDtypeStruct((_N * _UNITS,), jnp.float32),
        name="fused_embed_dense_sc",
        mesh=mesh,
        compiler_params=cp,
        scratch_types=[
            pltpu.VMEM((32,), jnp.float32),
            pltpu.VMEM((_JT * 8 * _IC,), jnp.int32),
            pltpu.VMEM((_JT * 8 * _IC,), jnp.int32),
            pltpu.VMEM((_JB * _IC * _UNITS,), jnp.float32),
            pltpu.VMEM((_JB * _IC * _UNITS,), jnp.float32),
            pltpu.SemaphoreType.DMA,
            pltpu.SemaphoreType.DMA,
            pltpu.SemaphoreType.DMA,
            pltpu.SemaphoreType.DMA,
        ],
    )(idxt, params)


def kernel(inputs, table, W, b):
    idxt = (inputs.astype(jnp.int32)
            .reshape(_B // 128, 128, _H // 8, 8)
            .transpose(2, 0, 3, 1)
            .reshape(-1))
    params = jnp.concatenate(
        [
            jnp.zeros((1,), jnp.float32),
            table.reshape(-1).astype(jnp.float32),
            W.reshape(-1).astype(jnp.float32),
            b.reshape(-1).astype(jnp.float32),
            jnp.zeros((31 - _VOCAB * _EMBED - _EMBED * _UNITS - _UNITS,), jnp.float32),
        ]
    )
    out = _sc_call(idxt, params)
    out = out.reshape(_H, _B // 128, _UNITS, 128).transpose(1, 3, 0, 2)
    return out.reshape(inputs.shape + (_UNITS,))


# speedup vs baseline: 1.3061x; 1.0659x over previous
import dataclasses

import jax
import jax.numpy as jnp
from jax import lax
from jax.experimental import pallas as pl
from jax.experimental.pallas import tpu as pltpu
from jax.experimental.pallas import tpu_sc as plsc

_VOCAB = 3
_EMBED = 2
_UNITS = 4
_LANES = 16

_B = 16384
_H = 200
_N = _B * _H
_NJG = 8
_NIQ = 4
_JB = _H // _NJG
_IB = _B // _NIQ
_IC = 256
_NCHUNKS = _IB // _IC
_JT = 4
_JSTRIDE = _B * _UNITS


def _body(idxt_hbm, params_hbm, out_hbm, params_v, idx0, idx1, out0, out1,
          si0, si1, so0, so1):
    c = lax.axis_index("c")
    s = lax.axis_index("s")
    wid = s * 2 + c
    jg = wid % _NJG
    iq = wid // _NJG
    j0 = jg * _JB
    i0q = iq * _IB

    pltpu.sync_copy(params_hbm, params_v)

    def splat(v):
        return jnp.full((_LANES,), v, jnp.int32)

    fused = []
    for v in range(_VOCAB):
        t0 = plsc.load_gather(params_v, [splat(1 + v * _EMBED + 0)])
        t1 = plsc.load_gather(params_v, [splat(1 + v * _EMBED + 1)])
        row = []
        for d in range(_UNITS):
            w0 = plsc.load_gather(params_v, [splat(7 + 0 * _UNITS + d)])
            w1 = plsc.load_gather(params_v, [splat(7 + 1 * _UNITS + d)])
            bd = plsc.load_gather(params_v, [splat(15 + d)])
            row.append(t0 * w0 + t1 * w1 + bd)
        fused.append(row)

    iota16 = lax.iota(jnp.int32, _LANES)
    fcols = [
        jnp.where(iota16 == 0, fused[0][d],
                  jnp.where(iota16 == 1, fused[1][d], fused[2][d]))
        for d in range(_UNITS)
    ]
    gd = lax.GatherDimensionNumbers(
        offset_dims=(), collapsed_slice_dims=(0,), start_index_map=(0,))

    def lut(fcol, idxv):
        return lax.gather(
            fcol, idxv[:, None], gd, (1,),
            mode=lax.GatherScatterMode.PROMISE_IN_BOUNDS)

    idx_bufs = (idx0, idx1)
    out_bufs = (out0, out1)
    isems = (si0, si1)
    osems = (so0, so1)

    jhi0 = j0 // 8

    def start_in(ci, b):
        it0 = iq * (_IB // 128) + ci * (_IC // 128)
        ib = idx_bufs[b]
        for a in range(_JT):
            for t in range(_IC // 128):
                src = idxt_hbm.at[pl.ds((jhi0 + a) * (_B * 8) + (it0 + t) * 1024, 1024)]
                pltpu.async_copy(
                    src,
                    ib.at[pl.ds((a * (_IC // 128) + t) * 1024, 1024)],
                    isems[b],
                )

    def start_out(ci, b):
        ob = out_bufs[b]
        o0 = iq * (_IB * _UNITS) + ci * (_IC * _UNITS)
        for jj in range(_JB):
            dst = out_hbm.at[pl.ds((j0 + jj) * _JSTRIDE + o0, _IC * _UNITS)]
            pltpu.async_copy(
                ob.at[pl.ds(jj * _IC * _UNITS, _IC * _UNITS)], dst, osems[b])

    def wait_in(b):
        pltpu.make_async_copy(
            idxt_hbm.at[pl.ds(0, _JT * 8 * _IC)], idx_bufs[b], isems[b]).wait()

    def wait_out(b):
        pltpu.make_async_copy(
            out_bufs[b], out_hbm.at[pl.ds(0, _JB * _IC * _UNITS)], osems[b]).wait()

    def compute(ci, b):
        ib = idx_bufs[b]
        ob = out_bufs[b]

        def jj_body(jj, carry):
            jfull = j0 + jj
            a = jfull // 8 - jhi0
            jlo = jfull % 8
            obase = jj * (_IC * _UNITS)
            for k in range(_IC // _LANES):
                t = k // 8
                ibase = ((a * (_IC // 128) + t) * 8 + jlo) * 128
                idxv = ib[pl.ds(ibase + (k % 8) * _LANES, _LANES)]
                off = t * 512 + (k % 8) * _LANES
                for d in range(_UNITS):
                    ob[pl.ds(obase + off + d * 128, _LANES)] = lut(fcols[d], idxv)
            return carry

        lax.fori_loop(0, _JB, jj_body, 0)

    start_in(0, 0)
    start_in(1, 1)
    for ci in range(2):
        wait_in(ci)
        compute(ci, ci)
        start_out(ci, ci)
        start_in(ci + 2, ci)

    def pair(g, carry):
        for b in range(2):
            ci = 2 * g + b
            wait_in(b)
            wait_out(b)
            compute(ci, b)
            start_out(ci, b)
            start_in(ci + 2, b)
        return carry

    lax.fori_loop(1, _NCHUNKS // 2 - 1, pair, 0)

    for ci in range(_NCHUNKS - 2, _NCHUNKS):
        b = ci % 2
        wait_in(b)
        wait_out(b)
        compute(ci, b)
        start_out(ci, b)
    wait_out(0)
    wait_out(1)


@jax.jit
def _sc_call(idxt, params):
    mesh = plsc.VectorSubcoreMesh(core_axis_name="c", subcore_axis_name="s")
    cp = pltpu.CompilerParams()
    if "needs_layout_passes" in pltpu.CompilerParams.__dataclass_fields__:
        cp = dataclasses.replace(cp, needs_layout_passes=False)
    return pl.kernel(
        _body,
        out_type=jax.ShapeDtypeStruct((_N * _UNITS,), jnp.float32),
        name="fused_embed_dense_sc",
        mesh=mesh,
        compiler_params=cp,
        scratch_types=[
            pltpu.VMEM((32,), jnp.float32),
            pltpu.VMEM((_JT * 8 * _IC,), jnp.int32),
            pltpu.VMEM((_JT * 8 * _IC,), jnp.int32),
            pltpu.VMEM((_JB * _IC * _UNITS,), jnp.float32),
            pltpu.VMEM((_JB * _IC * _UNITS,), jnp.float32),
            pltpu.SemaphoreType.DMA,
            pltpu.SemaphoreType.DMA,
            pltpu.SemaphoreType.DMA,
            pltpu.SemaphoreType.DMA,
        ],
    )(idxt, params)


def kernel(inputs, table, W, b):
    idxt = (inputs.astype(jnp.int32)
            .reshape(_B // 128, 128, _H // 8, 8)
            .transpose(2, 0, 3, 1)
            .reshape(-1))
    params = jnp.concatenate(
        [
            jnp.zeros((1,), jnp.float32),
            table.reshape(-1).astype(jnp.float32),
            W.reshape(-1).astype(jnp.float32),
            b.reshape(-1).astype(jnp.float32),
            jnp.zeros((31 - _VOCAB * _EMBED - _EMBED * _UNITS - _UNITS,), jnp.float32),
        ]
    )
    out = _sc_call(idxt, params)
    out = out.reshape(_H, _B // 128, _UNITS, 128).transpose(1, 3, 0, 2)
    return out.reshape(inputs.shape + (_UNITS,))


# speedup vs baseline: 1.6000x; 1.2250x over previous
import dataclasses

import jax
import jax.numpy as jnp
from jax import lax
from jax.experimental import pallas as pl
from jax.experimental.pallas import tpu as pltpu
from jax.experimental.pallas import tpu_sc as plsc

_VOCAB = 3
_EMBED = 2
_UNITS = 4
_LANES = 16

_B = 16384
_H = 200
_N = _B * _H
_NJG = 8
_NIQ = 4
_JB = _H // _NJG
_IB = _B // _NIQ
_IC = 256
_NCHUNKS = _IB // _IC
_JT = 4
_JSTRIDE = _B * _UNITS


def _body(idxt_hbm, params_hbm, out_hbm, params_v, idx0, idx1, out0, out1,
          si0, si1, so0, so1):
    c = lax.axis_index("c")
    s = lax.axis_index("s")
    wid = s * 2 + c
    jg = wid % _NJG
    iq = wid // _NJG
    j0 = jg * _JB
    i0q = iq * _IB

    pltpu.sync_copy(params_hbm, params_v)

    def splat(v):
        return jnp.full((_LANES,), v, jnp.int32)

    fused = []
    for v in range(_VOCAB):
        t0 = plsc.load_gather(params_v, [splat(1 + v * _EMBED + 0)])
        t1 = plsc.load_gather(params_v, [splat(1 + v * _EMBED + 1)])
        row = []
        for d in range(_UNITS):
            w0 = plsc.load_gather(params_v, [splat(7 + 0 * _UNITS + d)])
            w1 = plsc.load_gather(params_v, [splat(7 + 1 * _UNITS + d)])
            bd = plsc.load_gather(params_v, [splat(15 + d)])
            row.append(t0 * w0 + t1 * w1 + bd)
        fused.append(row)

    iota16 = lax.iota(jnp.int32, _LANES)
    fcols = [
        jnp.where(iota16 == 0, fused[0][d],
                  jnp.where(iota16 == 1, fused[1][d], fused[2][d]))
        for d in range(_UNITS)
    ]
    gd = lax.GatherDimensionNumbers(
        offset_dims=(), collapsed_slice_dims=(0,), start_index_map=(0,))

    def lut(fcol, idxv):
        return lax.gather(
            fcol, idxv[:, None], gd, (1,),
            mode=lax.GatherScatterMode.PROMISE_IN_BOUNDS)

    idx_bufs = (idx0, idx1)
    out_bufs = (out0, out1)
    isems = (si0, si1)
    osems = (so0, so1)

    jhi0 = j0 // 8

    def start_in(ci, b):
        it0 = iq * (_IB // 128) + ci * (_IC // 128)
        ib = idx_bufs[b]
        for a in range(_JT):
            for t in range(_IC // 128):
                src = idxt_hbm.at[pl.ds((jhi0 + a) * (_B * 8) + (it0 + t) * 1024, 1024)]
                pltpu.async_copy(
                    src,
                    ib.at[pl.ds((a * (_IC // 128) + t) * 1024, 1024)],
                    isems[b],
                )

    def start_out(ci, b):
        ob = out_bufs[b]
        o0 = iq * (_IB * _UNITS) + ci * (_IC * _UNITS)
        for jj in range(_JB):
            dst = out_hbm.at[pl.ds((j0 + jj) * _JSTRIDE + o0, _IC * _UNITS)]
            pltpu.async_copy(
                ob.at[pl.ds(jj * _IC * _UNITS, _IC * _UNITS)], dst, osems[b])

    def wait_in(b):
        pltpu.make_async_copy(
            idxt_hbm.at[pl.ds(0, _JT * 8 * _IC)], idx_bufs[b], isems[b]).wait()

    def wait_out(b):
        pltpu.make_async_copy(
            out_bufs[b], out_hbm.at[pl.ds(0, _JB * _IC * _UNITS)], osems[b]).wait()

    def compute(ci, b):
        ib = idx_bufs[b]
        ob = out_bufs[b]

        @plsc.parallel_loop(0, _JB, 1, unroll=2)
        def jj_body(jj):
            jfull = j0 + jj
            a = jfull // 8 - jhi0
            jlo = jfull % 8
            obase = jj * (_IC * _UNITS)
            for k in range(_IC // _LANES):
                t = k // 8
                ibase = ((a * (_IC // 128) + t) * 8 + jlo) * 128
                idxv = ib[pl.ds(ibase + (k % 8) * _LANES, _LANES)]
                off = t * 512 + (k % 8) * _LANES
                for d in range(_UNITS):
                    ob[pl.ds(obase + off + d * 128, _LANES)] = lut(fcols[d], idxv)

    start_in(0, 0)
    start_in(1, 1)
    for ci in range(2):
        wait_in(ci)
        compute(ci, ci)
        start_out(ci, ci)
        start_in(ci + 2, ci)

    def pair(g, carry):
        for b in range(2):
            ci = 2 * g + b
            wait_in(b)
            wait_out(b)
            compute(ci, b)
            start_out(ci, b)
            start_in(ci + 2, b)
        return carry

    lax.fori_loop(1, _NCHUNKS // 2 - 1, pair, 0)

    for ci in range(_NCHUNKS - 2, _NCHUNKS):
        b = ci % 2
        wait_in(b)
        wait_out(b)
        compute(ci, b)
        start_out(ci, b)
    wait_out(0)
    wait_out(1)


@jax.jit
def _sc_call(idxt, params):
    mesh = plsc.VectorSubcoreMesh(core_axis_name="c", subcore_axis_name="s")
    cp = pltpu.CompilerParams()
    if "needs_layout_passes" in pltpu.CompilerParams.__dataclass_fields__:
        cp = dataclasses.replace(cp, needs_layout_passes=False)
    return pl.kernel(
        _body,
        out_type=jax.ShapeDtypeStruct((_N * _UNITS,), jnp.float32),
        name="fused_embed_dense_sc",
        mesh=mesh,
        compiler_params=cp,
        scratch_types=[
            pltpu.VMEM((32,), jnp.float32),
            pltpu.VMEM((_JT * 8 * _IC,), jnp.int32),
            pltpu.VMEM((_JT * 8 * _IC,), jnp.int32),
            pltpu.VMEM((_JB * _IC * _UNITS,), jnp.float32),
            pltpu.VMEM((_JB * _IC * _UNITS,), jnp.float32),
            pltpu.SemaphoreType.DMA,
            pltpu.SemaphoreType.DMA,
            pltpu.SemaphoreType.DMA,
            pltpu.SemaphoreType.DMA,
        ],
    )(idxt, params)


def kernel(inputs, table, W, b):
    idxt = (inputs.astype(jnp.int32)
            .reshape(_B // 128, 128, _H // 8, 8)
            .transpose(2, 0, 3, 1)
            .reshape(-1))
    params = jnp.concatenate(
        [
            jnp.zeros((1,), jnp.float32),
            table.reshape(-1).astype(jnp.float32),
            W.reshape(-1).astype(jnp.float32),
            b.reshape(-1).astype(jnp.float32),
            jnp.zeros((31 - _VOCAB * _EMBED - _EMBED * _UNITS - _UNITS,), jnp.float32),
        ]
    )
    out = _sc_call(idxt, params)
    out = out.reshape(_H, _B // 128, _UNITS, 128).transpose(1, 3, 0, 2)
    return out.reshape(inputs.shape + (_UNITS,))


# speedup vs baseline: 1.6514x; 1.0321x over previous
import dataclasses

import jax
import jax.numpy as jnp
from jax import lax
from jax.experimental import pallas as pl
from jax.experimental.pallas import tpu as pltpu
from jax.experimental.pallas import tpu_sc as plsc

_VOCAB = 3
_EMBED = 2
_UNITS = 4
_LANES = 16

_B = 16384
_H = 200
_N = _B * _H
_NJG = 8
_NIQ = 4
_JB = _H // _NJG
_IB = _B // _NIQ
_IC = 256
_NCHUNKS = _IB // _IC
_JT = 4
_JSTRIDE = _B * _UNITS


def _body(idxt_hbm, params_hbm, out_hbm, params_v, idx0, idx1, out0, out1,
          si0, si1, so0, so1):
    c = lax.axis_index("c")
    s = lax.axis_index("s")
    wid = s * 2 + c
    jg = wid % _NJG
    iq = wid // _NJG
    j0 = jg * _JB
    i0q = iq * _IB

    pltpu.sync_copy(params_hbm, params_v)

    def splat(v):
        return jnp.full((_LANES,), v, jnp.int32)

    fused = []
    for v in range(_VOCAB):
        t0 = plsc.load_gather(params_v, [splat(1 + v * _EMBED + 0)])
        t1 = plsc.load_gather(params_v, [splat(1 + v * _EMBED + 1)])
        row = []
        for d in range(_UNITS):
            w0 = plsc.load_gather(params_v, [splat(7 + 0 * _UNITS + d)])
            w1 = plsc.load_gather(params_v, [splat(7 + 1 * _UNITS + d)])
            bd = plsc.load_gather(params_v, [splat(15 + d)])
            row.append(t0 * w0 + t1 * w1 + bd)
        fused.append(row)

    iota16 = lax.iota(jnp.int32, _LANES)
    fcols = [
        jnp.where(iota16 == 0, fused[0][d],
                  jnp.where(iota16 == 1, fused[1][d], fused[2][d]))
        for d in range(_UNITS)
    ]
    gd = lax.GatherDimensionNumbers(
        offset_dims=(), collapsed_slice_dims=(0,), start_index_map=(0,))

    def lut(fcol, idxv):
        return lax.gather(
            fcol, idxv[:, None], gd, (1,),
            mode=lax.GatherScatterMode.PROMISE_IN_BOUNDS)

    idx_bufs = (idx0, idx1)
    out_bufs = (out0, out1)
    isems = (si0, si1)
    osems = (so0, so1)

    jhi0 = j0 // 8

    def start_in(ci, b):
        it0 = iq * (_IB // 128) + ci * (_IC // 128)
        ib = idx_bufs[b]
        for a in range(_JT):
            for t in range(_IC // 128):
                src = idxt_hbm.at[pl.ds((jhi0 + a) * (_B * 8) + (it0 + t) * 1024, 1024)]
                pltpu.async_copy(
                    src,
                    ib.at[pl.ds((a * (_IC // 128) + t) * 1024, 1024)],
                    isems[b],
                )

    def start_out(ci, b):
        ob = out_bufs[b]
        o0 = iq * (_IB * _UNITS) + ci * (_IC * _UNITS)
        for jj in range(_JB):
            dst = out_hbm.at[pl.ds((j0 + jj) * _JSTRIDE + o0, _IC * _UNITS)]
            pltpu.async_copy(
                ob.at[pl.ds(jj * _IC * _UNITS, _IC * _UNITS)], dst, osems[b])

    def wait_in(b):
        pltpu.make_async_copy(
            idxt_hbm.at[pl.ds(0, _JT * 8 * _IC)], idx_bufs[b], isems[b]).wait()

    def wait_out(b):
        pltpu.make_async_copy(
            out_bufs[b], out_hbm.at[pl.ds(0, _JB * _IC * _UNITS)], osems[b]).wait()

    def compute(ci, b):
        ib = idx_bufs[b]
        ob = out_bufs[b]

        @plsc.parallel_loop(0, _JB, 1, unroll=4)
        def jj_body(jj):
            jfull = j0 + jj
            a = jfull // 8 - jhi0
            jlo = jfull % 8
            obase = jj * (_IC * _UNITS)
            for k in range(_IC // _LANES):
                t = k // 8
                ibase = ((a * (_IC // 128) + t) * 8 + jlo) * 128
                idxv = ib[pl.ds(ibase + (k % 8) * _LANES, _LANES)]
                off = t * 512 + (k % 8) * _LANES
                for d in range(_UNITS):
                    ob[pl.ds(obase + off + d * 128, _LANES)] = lut(fcols[d], idxv)

    start_in(0, 0)
    start_in(1, 1)
    for ci in range(2):
        wait_in(ci)
        compute(ci, ci)
        start_out(ci, ci)
        start_in(ci + 2, ci)

    def pair(g, carry):
        for b in range(2):
            ci = 2 * g + b
            wait_in(b)
            wait_out(b)
            compute(ci, b)
            start_out(ci, b)
            start_in(ci + 2, b)
        return carry

    lax.fori_loop(1, _NCHUNKS // 2 - 1, pair, 0)

    for ci in range(_NCHUNKS - 2, _NCHUNKS):
        b = ci % 2
        wait_in(b)
        wait_out(b)
        compute(ci, b)
        start_out(ci, b)
    wait_out(0)
    wait_out(1)


@jax.jit
def _sc_call(idxt, params):
    mesh = plsc.VectorSubcoreMesh(core_axis_name="c", subcore_axis_name="s")
    cp = pltpu.CompilerParams()
    if "needs_layout_passes" in pltpu.CompilerParams.__dataclass_fields__:
        cp = dataclasses.replace(cp, needs_layout_passes=False)
    return pl.kernel(
        _body,
        out_type=jax.ShapeDtypeStruct((_N * _UNITS,), jnp.float32),
        name="fused_embed_dense_sc",
        mesh=mesh,
        compiler_params=cp,
        scratch_types=[
            pltpu.VMEM((32,), jnp.float32),
            pltpu.VMEM((_JT * 8 * _IC,), jnp.int32),
            pltpu.VMEM((_JT * 8 * _IC,), jnp.int32),
            pltpu.VMEM((_JB * _IC * _UNITS,), jnp.float32),
            pltpu.VMEM((_JB * _IC * _UNITS,), jnp.float32),
            pltpu.SemaphoreType.DMA,
            pltpu.SemaphoreType.DMA,
            pltpu.SemaphoreType.DMA,
            pltpu.SemaphoreType.DMA,
        ],
    )(idxt, params)


def kernel(inputs, table, W, b):
    idxt = (inputs.astype(jnp.int32)
            .reshape(_B // 128, 128, _H // 8, 8)
            .transpose(2, 0, 3, 1)
            .reshape(-1))
    params = jnp.concatenate(
        [
            jnp.zeros((1,), jnp.float32),
            table.reshape(-1).astype(jnp.float32),
            W.reshape(-1).astype(jnp.float32),
            b.reshape(-1).astype(jnp.float32),
            jnp.zeros((31 - _VOCAB * _EMBED - _EMBED * _UNITS - _UNITS,), jnp.float32),
        ]
    )
    out = _sc_call(idxt, params)
    out = out.reshape(_H, _B // 128, _UNITS, 128).transpose(1, 3, 0, 2)
    return out.reshape(inputs.shape + (_UNITS,))


# speedup vs baseline: 1.6669x; 1.0094x over previous
import dataclasses

import jax
import jax.numpy as jnp
from jax import lax
from jax.experimental import pallas as pl
from jax.experimental.pallas import tpu as pltpu
from jax.experimental.pallas import tpu_sc as plsc

_VOCAB = 3
_EMBED = 2
_UNITS = 4
_LANES = 16

_B = 16384
_H = 200
_N = _B * _H
_NJG = 8
_NIQ = 4
_JB = _H // _NJG
_IB = _B // _NIQ
_IC = 256
_NCHUNKS = _IB // _IC
_JT = 4
_JSTRIDE = _B * _UNITS


def _body(idxt_hbm, params_hbm, out_hbm, params_v, idx0, idx1, out0, out1,
          si0, si1, so0, so1):
    c = lax.axis_index("c")
    s = lax.axis_index("s")
    wid = s * 2 + c
    jg = wid % _NJG
    iq = wid // _NJG
    j0 = jg * _JB
    i0q = iq * _IB

    pltpu.sync_copy(params_hbm, params_v)

    def splat(v):
        return jnp.full((_LANES,), v, jnp.int32)

    fused = []
    for v in range(_VOCAB):
        t0 = plsc.load_gather(params_v, [splat(1 + v * _EMBED + 0)])
        t1 = plsc.load_gather(params_v, [splat(1 + v * _EMBED + 1)])
        row = []
        for d in range(_UNITS):
            w0 = plsc.load_gather(params_v, [splat(7 + 0 * _UNITS + d)])
            w1 = plsc.load_gather(params_v, [splat(7 + 1 * _UNITS + d)])
            bd = plsc.load_gather(params_v, [splat(15 + d)])
            row.append(t0 * w0 + t1 * w1 + bd)
        fused.append(row)

    iota16 = lax.iota(jnp.int32, _LANES)
    fcols = [
        jnp.where(iota16 == 0, fused[0][d],
                  jnp.where(iota16 == 1, fused[1][d], fused[2][d]))
        for d in range(_UNITS)
    ]
    gd = lax.GatherDimensionNumbers(
        offset_dims=(), collapsed_slice_dims=(0,), start_index_map=(0,))

    def lut(fcol, idxv):
        return lax.gather(
            fcol, idxv[:, None], gd, (1,),
            mode=lax.GatherScatterMode.PROMISE_IN_BOUNDS)

    idx_bufs = (idx0, idx1)
    out_bufs = (out0, out1)
    isems = (si0, si1)
    osems = (so0, so1)

    jhi0 = j0 // 8

    def start_in(ci, b):
        it0 = iq * (_IB // 128) + ci * (_IC // 128)
        ib = idx_bufs[b]
        for a in range(_JT):
            for t in range(_IC // 128):
                src = idxt_hbm.at[pl.ds((jhi0 + a) * (_B * 8) + (it0 + t) * 1024, 1024)]
                pltpu.async_copy(
                    src,
                    ib.at[pl.ds((a * (_IC // 128) + t) * 1024, 1024)],
                    isems[b],
                )

    def start_out(ci, b):
        ob = out_bufs[b]
        o0 = iq * (_IB * _UNITS) + ci * (_IC * _UNITS)
        for jj in range(_JB):
            dst = out_hbm.at[pl.ds((j0 + jj) * _JSTRIDE + o0, _IC * _UNITS)]
            pltpu.async_copy(
                ob.at[pl.ds(jj * _IC * _UNITS, _IC * _UNITS)], dst, osems[b])

    def wait_in(b):
        pltpu.make_async_copy(
            idxt_hbm.at[pl.ds(0, _JT * 8 * _IC)], idx_bufs[b], isems[b]).wait()

    def wait_out(b):
        pltpu.make_async_copy(
            out_bufs[b], out_hbm.at[pl.ds(0, _JB * _IC * _UNITS)], osems[b]).wait()

    def compute(ci, b):
        ib = idx_bufs[b]
        ob = out_bufs[b]

        @plsc.parallel_loop(0, _JB, 1, unroll=5)
        def jj_body(jj):
            jfull = j0 + jj
            a = jfull // 8 - jhi0
            jlo = jfull % 8
            obase = jj * (_IC * _UNITS)
            for k in range(_IC // _LANES):
                t = k // 8
                ibase = ((a * (_IC // 128) + t) * 8 + jlo) * 128
                idxv = ib[pl.ds(ibase + (k % 8) * _LANES, _LANES)]
                off = t * 512 + (k % 8) * _LANES
                for d in range(_UNITS):
                    ob[pl.ds(obase + off + d * 128, _LANES)] = lut(fcols[d], idxv)

    start_in(0, 0)
    start_in(1, 1)
    for ci in range(2):
        wait_in(ci)
        compute(ci, ci)
        start_out(ci, ci)
        start_in(ci + 2, ci)

    def pair(g, carry):
        for b in range(2):
            ci = 2 * g + b
            wait_in(b)
            wait_out(b)
            compute(ci, b)
            start_out(ci, b)
            start_in(ci + 2, b)
        return carry

    lax.fori_loop(1, _NCHUNKS // 2 - 1, pair, 0)

    for ci in range(_NCHUNKS - 2, _NCHUNKS):
        b = ci % 2
        wait_in(b)
        wait_out(b)
        compute(ci, b)
        start_out(ci, b)
    wait_out(0)
    wait_out(1)


@jax.jit
def _sc_call(idxt, params):
    mesh = plsc.VectorSubcoreMesh(core_axis_name="c", subcore_axis_name="s")
    cp = pltpu.CompilerParams()
    if "needs_layout_passes" in pltpu.CompilerParams.__dataclass_fields__:
        cp = dataclasses.replace(cp, needs_layout_passes=False)
    return pl.kernel(
        _body,
        out_type=jax.ShapeDtypeStruct((_N * _UNITS,), jnp.float32),
        name="fused_embed_dense_sc",
        mesh=mesh,
        compiler_params=cp,
        scratch_types=[
            pltpu.VMEM((32,), jnp.float32),
            pltpu.VMEM((_JT * 8 * _IC,), jnp.int32),
            pltpu.VMEM((_JT * 8 * _IC,), jnp.int32),
            pltpu.VMEM((_JB * _IC * _UNITS,), jnp.float32),
            pltpu.VMEM((_JB * _IC * _UNITS,), jnp.float32),
            pltpu.SemaphoreType.DMA,
            pltpu.SemaphoreType.DMA,
            pltpu.SemaphoreType.DMA,
            pltpu.SemaphoreType.DMA,
        ],
    )(idxt, params)


def kernel(inputs, table, W, b):
    idxt = (inputs.astype(jnp.int32)
            .reshape(_B // 128, 128, _H // 8, 8)
            .transpose(2, 0, 3, 1)
            .reshape(-1))
    params = jnp.concatenate(
        [
            jnp.zeros((1,), jnp.float32),
            table.reshape(-1).astype(jnp.float32),
            W.reshape(-1).astype(jnp.float32),
            b.reshape(-1).astype(jnp.float32),
            jnp.zeros((31 - _VOCAB * _EMBED - _EMBED * _UNITS - _UNITS,), jnp.float32),
        ]
    )
    out = _sc_call(idxt, params)
    out = out.reshape(_H, _B // 128, _UNITS, 128).transpose(1, 3, 0, 2)
    return out.reshape(inputs.shape + (_UNITS,))
